# baseline (speedup 1.0000x reference)
import jax
import jax.numpy as jnp
from jax.experimental import pallas as pl
from jax.experimental.pallas import tpu as pltpu

_B = 256
_N = 4096
_D = 256
_M = 32
_NT = _D // _M
_NBUF = 4


def _opd_kernel(pos_ref, obs_t_ref, poly_ref, graph_ref, ap_ref, ag_ref,
                sp_ref, sg_ref, out_ref, c_ref, s_all_ref, obuf_ref, sems):
    i = pl.program_id(0)
    pf = ap_ref.shape[1]

    @pl.when(i == 0)
    def _stage_a():
        px = pos_ref[:, 0:1]
        py = pos_ref[:, 1:2]
        ox = obs_t_ref[0:1, :]
        oy = obs_t_ref[1:2, :]
        d = (px - ox) ** 2 + (py - oy) ** 2
        md = jnp.min(d, axis=1, keepdims=True)
        iota = jax.lax.broadcasted_iota(jnp.int32, (_B, _N), 1)
        idx = jnp.min(jnp.where(d == md, iota, _N), axis=1, keepdims=True)
        onehot = (iota == idx).astype(jnp.float32)
        gp = jnp.dot(onehot, poly_ref[...], preferred_element_type=jnp.float32)
        gg = jnp.dot(onehot, graph_ref[...], preferred_element_type=jnp.float32)
        cp = jnp.dot(gp, ap_ref[...], preferred_element_type=jnp.float32)
        cg = jnp.dot(gg, ag_ref[...], preferred_element_type=jnp.float32)
        c_ref[...] = jnp.concatenate([cp, cg], axis=1)
        s_all_ref[0:pf] = sp_ref[...]
        s_all_ref[pf:] = sg_ref[...]

    slot = jax.lax.rem(i, _NBUF)

    def _copy(step, buf):
        return pltpu.make_async_copy(
            obuf_ref.at[buf], out_ref.at[:, pl.ds(step * _M, _M), :],
            sems.at[buf])

    @pl.when(i >= _NBUF)
    def _wait_prev():
        _copy(i - _NBUF, slot).wait()

    s2 = s_all_ref[:, pl.ds(i * _M, _M), :].reshape(2 * pf, _M * _D)
    r = jnp.dot(c_ref[...], s2, preferred_element_type=jnp.float32)
    obuf_ref[slot] = r.reshape(_B, _M, _D)
    _copy(i, slot).start()

    @pl.when(i == _NT - 1)
    def _drain():
        for back in range(_NBUF - 1, -1, -1):
            _copy(i - back, jax.lax.rem(i - back, _NBUF)).wait()


def kernel(positions, obs_pos, poly_dic, graph_dic, S_poly, S_graph,
           alpha_poly, alpha_graph):
    pf = alpha_poly.shape[1]
    gf = alpha_graph.shape[1]
    k = pf + gf
    obs_t = obs_pos.T

    opd_maps = pl.pallas_call(
        _opd_kernel,
        grid=(_NT,),
        in_specs=[
            pl.BlockSpec((_B, 2), lambda i: (0, 0)),
            pl.BlockSpec((2, _N), lambda i: (0, 0)),
            pl.BlockSpec(poly_dic.shape, lambda i: (0, 0)),
            pl.BlockSpec(graph_dic.shape, lambda i: (0, 0)),
            pl.BlockSpec(alpha_poly.shape, lambda i: (0, 0)),
            pl.BlockSpec(alpha_graph.shape, lambda i: (0, 0)),
            pl.BlockSpec((pf, _D, _D), lambda i: (0, 0, 0)),
            pl.BlockSpec((gf, _D, _D), lambda i: (0, 0, 0)),
        ],
        out_specs=pl.BlockSpec(memory_space=pl.ANY),
        out_shape=jax.ShapeDtypeStruct((_B, _D, _D), jnp.float32),
        scratch_shapes=[
            pltpu.VMEM((_B, k), jnp.float32),
            pltpu.VMEM((k, _D, _D), jnp.float32),
            pltpu.VMEM((_NBUF, _B, _M, _D), jnp.float32),
            pltpu.SemaphoreType.DMA((_NBUF,)),
        ],
    )(positions, obs_t, poly_dic, graph_dic, alpha_poly, alpha_graph,
      S_poly, S_graph)

    return (opd_maps, alpha_graph)


# speedup vs baseline: 1.0035x; 1.0035x over previous
import jax
import jax.numpy as jnp
from jax.experimental import pallas as pl
from jax.experimental.pallas import tpu as pltpu

_B = 256
_N = 4096
_D = 256
_M = 32
_NT = _D // _M
_NBUF = 3


def _opd_kernel(pos_ref, obs_t_ref, poly_ref, graph_ref, ap_ref, ag_ref,
                sp_ref, sg_ref, out_ref, c_ref, s_all_ref, obuf_ref, sems):
    i = pl.program_id(0)
    pf = ap_ref.shape[1]

    @pl.when(i == 0)
    def _stage_a():
        px = pos_ref[:, 0:1]
        py = pos_ref[:, 1:2]
        ox = obs_t_ref[0:1, :]
        oy = obs_t_ref[1:2, :]
        d = (px - ox) ** 2 + (py - oy) ** 2
        md = jnp.min(d, axis=1, keepdims=True)
        iota = jax.lax.broadcasted_iota(jnp.int32, (_B, _N), 1)
        idx = jnp.min(jnp.where(d == md, iota, _N), axis=1, keepdims=True)
        onehot = (iota == idx).astype(jnp.float32)
        gp = jnp.dot(onehot, poly_ref[...], preferred_element_type=jnp.float32)
        gg = jnp.dot(onehot, graph_ref[...], preferred_element_type=jnp.float32)
        cp = jnp.dot(gp, ap_ref[...], preferred_element_type=jnp.float32)
        cg = jnp.dot(gg, ag_ref[...], preferred_element_type=jnp.float32)
        c_ref[...] = jnp.concatenate([cp, cg], axis=1)
        s_all_ref[0:pf] = sp_ref[...]
        s_all_ref[pf:] = sg_ref[...]

    slot = jax.lax.rem(i, _NBUF)

    def _copy(step, buf):
        return pltpu.make_async_copy(
            obuf_ref.at[buf], out_ref.at[:, pl.ds(step * _M, _M), :],
            sems.at[buf])

    @pl.when(i >= _NBUF)
    def _wait_prev():
        _copy(i - _NBUF, slot).wait()

    s2 = s_all_ref[:, pl.ds(i * _M, _M), :].reshape(2 * pf, _M * _D)
    r = jnp.dot(c_ref[...], s2, preferred_element_type=jnp.float32)
    obuf_ref[slot] = r.reshape(_B, _M, _D)
    _copy(i, slot).start()

    @pl.when(i == _NT - 1)
    def _drain():
        for back in range(_NBUF - 1, -1, -1):
            _copy(i - back, jax.lax.rem(i - back, _NBUF)).wait()


def kernel(positions, obs_pos, poly_dic, graph_dic, S_poly, S_graph,
           alpha_poly, alpha_graph):
    pf = alpha_poly.shape[1]
    gf = alpha_graph.shape[1]
    k = pf + gf
    obs_t = obs_pos.T

    opd_maps = pl.pallas_call(
        _opd_kernel,
        grid=(_NT,),
        in_specs=[
            pl.BlockSpec((_B, 2), lambda i: (0, 0)),
            pl.BlockSpec((2, _N), lambda i: (0, 0)),
            pl.BlockSpec(poly_dic.shape, lambda i: (0, 0)),
            pl.BlockSpec(graph_dic.shape, lambda i: (0, 0)),
            pl.BlockSpec(alpha_poly.shape, lambda i: (0, 0)),
            pl.BlockSpec(alpha_graph.shape, lambda i: (0, 0)),
            pl.BlockSpec((pf, _D, _D), lambda i: (0, 0, 0)),
            pl.BlockSpec((gf, _D, _D), lambda i: (0, 0, 0)),
        ],
        out_specs=pl.BlockSpec(memory_space=pl.ANY),
        out_shape=jax.ShapeDtypeStruct((_B, _D, _D), jnp.float32),
        scratch_shapes=[
            pltpu.VMEM((_B, k), jnp.float32),
            pltpu.VMEM((k, _D, _D), jnp.float32),
            pltpu.VMEM((_NBUF, _B, _M, _D), jnp.float32),
            pltpu.SemaphoreType.DMA((_NBUF,)),
        ],
    )(positions, obs_t, poly_dic, graph_dic, alpha_poly, alpha_graph,
      S_poly, S_graph)

    return (opd_maps, alpha_graph)
